# Initial kernel scaffold; baseline (speedup 1.0000x reference)
import jax
import jax.numpy as jnp
from jax.experimental import pallas as pl


def kernel(coords, features):
    raise NotImplementedError("write your pallas kernel here")


# speedup vs baseline: 4.1906x; 4.1906x over previous
import functools

import jax
import jax.numpy as jnp
from jax import lax
from jax.experimental import pallas as pl
from jax.experimental.pallas import tpu as pltpu
from jax.experimental.pallas import tpu_sc as plsc

_B, _L, _C = 8, 16384, 128
_S = 256
_N = _B * _L
_NC, _NS = 2, 16
_CHUNK = 8192
_PASSES_PER_SC = _N // _CHUNK // _NC
_BLK = 128
_JUNK = _CHUNK
_PAD = 2 * _BLK


def _sc_body(feats, permr, segr, metar, zrows, out,
             acc, zv, pidx, sidx, dloc, rows, mvec, sem):
    c = lax.axis_index("c")
    t = lax.axis_index("s")
    rows_per_tile = _CHUNK // _NS

    pltpu.sync_copy(zrows, zv)
    pltpu.sync_copy(metar.at[c], mvec)
    mv = mvec[...]

    for p in range(_PASSES_PER_SC):
        base = (c * _PASSES_PER_SC + p) * _CHUNK

        for q in range(rows_per_tile // _BLK):
            pltpu.sync_copy(zv, acc.at[pl.ds(t * rows_per_tile + q * _BLK, _BLK)])

        @pl.when(t == _NS - 1)
        def _zero_junk():
            pltpu.sync_copy(zv.at[pl.ds(0, 8)], acc.at[pl.ds(_CHUNK, 8)])

        plsc.subcore_barrier()

        pos0 = mv[p]
        nblk = mv[_PASSES_PER_SC + p]

        nsteps = jnp.maximum(nblk - t + _NS - 1, 0) // _NS

        def _step(i, carry):
            j = t + i * _NS
            pos = pl.multiple_of(pos0 + j * _BLK, 8)
            idx_cp = pltpu.async_copy(permr.at[pl.ds(pos, _BLK)], pidx, sem)
            seg_cp = pltpu.async_copy(segr.at[pl.ds(pos, _BLK)], sidx, sem)
            idx_cp.wait()
            seg_cp.wait()
            for k in range(_BLK // 16):
                sv = sidx[pl.ds(k * 16, 16)] - base
                ok = (sv >= 0) & (sv < _CHUNK)
                dloc[pl.ds(k * 16, 16)] = jnp.where(ok, sv, jnp.int32(_JUNK))
            pltpu.async_copy(feats.at[pidx], rows, sem).wait()
            pltpu.async_copy(rows, acc.at[dloc], sem, add=True).wait()
            return carry

        lax.fori_loop(0, nsteps, _step, jnp.int32(0))
        plsc.subcore_barrier()

        obase = pl.multiple_of(base + t * rows_per_tile, 8)
        pltpu.sync_copy(acc.at[pl.ds(t * rows_per_tile, rows_per_tile)],
                        out.at[pl.ds(obase, rows_per_tile)])
        plsc.subcore_barrier()


_mesh = plsc.VectorSubcoreMesh(core_axis_name="c", subcore_axis_name="s")

_sc_call = pl.kernel(
    _sc_body,
    out_type=jax.ShapeDtypeStruct((_N, _C), jnp.float32),
    mesh=_mesh,
    scratch_types=[
        pltpu.VMEM_SHARED((_CHUNK + 8, _C), jnp.float32),
        pltpu.VMEM((_BLK, _C), jnp.float32),
        pltpu.VMEM((_BLK,), jnp.int32),
        pltpu.VMEM((_BLK,), jnp.int32),
        pltpu.VMEM((_BLK,), jnp.int32),
        pltpu.VMEM((_BLK, _C), jnp.float32),
        pltpu.VMEM((16,), jnp.int32),
        pltpu.SemaphoreType.DMA,
    ],
)


@jax.jit
def kernel(coords, features):
    ci = coords.astype(jnp.int32)
    lin = (ci[..., 0] * _S + ci[..., 1]) * _S + ci[..., 2]
    keys = (jnp.arange(_B, dtype=jnp.int32)[:, None] * (_S ** 3) + lin)
    keys = keys.reshape(-1)
    feats = features.reshape(_N, _C)

    sorted_keys, perm = lax.sort_key_val(keys, jnp.arange(_N, dtype=jnp.int32))
    flags = jnp.concatenate([
        jnp.zeros((1,), jnp.int32),
        (sorted_keys[1:] != sorted_keys[:-1]).astype(jnp.int32),
    ])
    seg = jnp.cumsum(flags, dtype=jnp.int32)

    targets = jnp.arange(_N // _CHUNK + 1, dtype=jnp.int32) * _CHUNK
    bounds = jnp.searchsorted(seg, targets, side="left").astype(jnp.int32)
    pos0 = (bounds[:-1] // 8) * 8
    nblk = jnp.maximum(0, (bounds[1:] - pos0 + _BLK - 1) // _BLK)
    meta = jnp.concatenate([
        pos0.reshape(_NC, _PASSES_PER_SC),
        nblk.reshape(_NC, _PASSES_PER_SC),
    ], axis=1)

    perm_pad = jnp.concatenate([perm, jnp.zeros((_PAD,), jnp.int32)])
    seg_pad = jnp.concatenate([seg, jnp.full((_PAD,), 2 ** 30, jnp.int32)])
    zeros_block = jnp.zeros((_BLK, _C), jnp.float32)

    return _sc_call(feats, perm_pad, seg_pad, meta, zeros_block)


# speedup vs baseline: 4.7620x; 1.1364x over previous
import functools

import jax
import jax.numpy as jnp
from jax import lax
from jax.experimental import pallas as pl
from jax.experimental.pallas import tpu as pltpu
from jax.experimental.pallas import tpu_sc as plsc

_B, _L, _C = 8, 16384, 128
_S = 256
_N = _B * _L
_NC, _NS = 2, 16
_CHUNK = 8192
_PASSES_PER_SC = _N // _CHUNK // _NC
_BLK = 128
_JUNK = _CHUNK
_PAD = 2 * _BLK


def _sc_body(feats, permr, segr, metar, zrows, out,
             acc, zv,
             pidx0, sidx0, dloc0, rows0,
             pidx1, sidx1, dloc1, rows1,
             mvec, semi0, semg0, sems0, semi1, semg1, sems1):
    c = lax.axis_index("c")
    t = lax.axis_index("s")
    rows_per_tile = _CHUNK // _NS

    pltpu.sync_copy(zrows, zv)
    pltpu.sync_copy(metar.at[c], mvec)
    mv = mvec[...]

    bufs = ((pidx0, sidx0, dloc0, rows0, semi0, semg0, sems0),
            (pidx1, sidx1, dloc1, rows1, semi1, semg1, sems1))

    for p in range(_PASSES_PER_SC):
        base = (c * _PASSES_PER_SC + p) * _CHUNK

        for q in range(rows_per_tile // _BLK):
            pltpu.sync_copy(zv, acc.at[pl.ds(t * rows_per_tile + q * _BLK, _BLK)])

        @pl.when(t == _NS - 1)
        def _zero_junk():
            pltpu.sync_copy(zv.at[pl.ds(0, 8)], acc.at[pl.ds(_CHUNK, 8)])

        plsc.subcore_barrier()

        pos0 = mv[p]
        nblk = mv[_PASSES_PER_SC + p]

        nsteps = jnp.maximum(nblk - t + _NS - 1, 0) // _NS

        def _blockpos(ib):
            return pl.multiple_of(pos0 + (t + ib * _NS) * _BLK, 8)

        def _issue_idx(ib, parity):
            pi, si, _, _, smi, _, _ = bufs[parity]
            pos = _blockpos(ib)
            pltpu.async_copy(permr.at[pl.ds(pos, _BLK)], pi, smi)
            pltpu.async_copy(segr.at[pl.ds(pos, _BLK)], si, smi)

        def _wait_idx(ib, parity):
            pi, si, _, _, smi, _, _ = bufs[parity]
            pos = _blockpos(ib)
            pltpu.make_async_copy(permr.at[pl.ds(pos, _BLK)], pi, smi).wait()
            pltpu.make_async_copy(segr.at[pl.ds(pos, _BLK)], si, smi).wait()

        for parity in (0, 1):
            @pl.when(parity < nsteps)
            def _pro(parity=parity):
                _issue_idx(jnp.int32(parity), parity)

        def _iter(i, carry):
            for parity in (0, 1):
                ib = 2 * i + parity

                @pl.when(ib < nsteps)
                def _do(ib=ib, parity=parity):
                    pi, si, dl, rw, smi, smg, sms = bufs[parity]
                    _wait_idx(ib, parity)
                    @pl.when(ib >= 2)
                    def _ws():
                        pltpu.make_async_copy(rw, acc.at[dl], sms).wait()
                    for k in range(_BLK // 16):
                        sv = si[pl.ds(k * 16, 16)] - base
                        ok = (sv >= 0) & (sv < _CHUNK)
                        dl[pl.ds(k * 16, 16)] = jnp.where(ok, sv, jnp.int32(_JUNK))
                    g = pltpu.async_copy(feats.at[pi], rw, smg)
                    g.wait()

                    @pl.when(ib + 2 < nsteps)
                    def _ni():
                        _issue_idx(ib + 2, parity)
                    pltpu.async_copy(rw, acc.at[dl], sms, add=True)
            return carry

        lax.fori_loop(0, (nsteps + 1) // 2, _iter, jnp.int32(0))

        for parity in (0, 1):
            @pl.when(nsteps > parity)
            def _drain(parity=parity):
                _, _, dl, rw, _, _, sms = bufs[parity]
                pltpu.make_async_copy(rw, acc.at[dl], sms).wait()

        plsc.subcore_barrier()

        obase = pl.multiple_of(base + t * rows_per_tile, 8)
        pltpu.sync_copy(acc.at[pl.ds(t * rows_per_tile, rows_per_tile)],
                        out.at[pl.ds(obase, rows_per_tile)])
        plsc.subcore_barrier()


_mesh = plsc.VectorSubcoreMesh(core_axis_name="c", subcore_axis_name="s")

_sc_call = pl.kernel(
    _sc_body,
    out_type=jax.ShapeDtypeStruct((_N, _C), jnp.float32),
    mesh=_mesh,
    scratch_types=[
        pltpu.VMEM_SHARED((_CHUNK + 8, _C), jnp.float32),
        pltpu.VMEM((_BLK, _C), jnp.float32),
        pltpu.VMEM((_BLK,), jnp.int32),
        pltpu.VMEM((_BLK,), jnp.int32),
        pltpu.VMEM((_BLK,), jnp.int32),
        pltpu.VMEM((_BLK, _C), jnp.float32),
        pltpu.VMEM((_BLK,), jnp.int32),
        pltpu.VMEM((_BLK,), jnp.int32),
        pltpu.VMEM((_BLK,), jnp.int32),
        pltpu.VMEM((_BLK, _C), jnp.float32),
        pltpu.VMEM((16,), jnp.int32),
        pltpu.SemaphoreType.DMA,
        pltpu.SemaphoreType.DMA,
        pltpu.SemaphoreType.DMA,
        pltpu.SemaphoreType.DMA,
        pltpu.SemaphoreType.DMA,
        pltpu.SemaphoreType.DMA,
    ],
)


@jax.jit
def kernel(coords, features):
    ci = coords.astype(jnp.int32)
    lin = (ci[..., 0] * _S + ci[..., 1]) * _S + ci[..., 2]
    keys = (jnp.arange(_B, dtype=jnp.int32)[:, None] * (_S ** 3) + lin)
    keys = keys.reshape(-1)
    feats = features.reshape(_N, _C)

    sorted_keys, perm = lax.sort_key_val(keys, jnp.arange(_N, dtype=jnp.int32))
    flags = jnp.concatenate([
        jnp.zeros((1,), jnp.int32),
        (sorted_keys[1:] != sorted_keys[:-1]).astype(jnp.int32),
    ])
    seg = jnp.cumsum(flags, dtype=jnp.int32)

    targets = jnp.arange(_N // _CHUNK + 1, dtype=jnp.int32) * _CHUNK
    bounds = jnp.searchsorted(seg, targets, side="left").astype(jnp.int32)
    pos0 = (bounds[:-1] // 8) * 8
    nblk = jnp.maximum(0, (bounds[1:] - pos0 + _BLK - 1) // _BLK)
    meta = jnp.concatenate([
        pos0.reshape(_NC, _PASSES_PER_SC),
        nblk.reshape(_NC, _PASSES_PER_SC),
    ], axis=1)

    perm_pad = jnp.concatenate([perm, jnp.zeros((_PAD,), jnp.int32)])
    seg_pad = jnp.concatenate([seg, jnp.full((_PAD,), 2 ** 30, jnp.int32)])
    zeros_block = jnp.zeros((_BLK, _C), jnp.float32)

    return _sc_call(feats, perm_pad, seg_pad, meta, zeros_block)


# speedup vs baseline: 5.0908x; 1.0690x over previous
import functools

import jax
import jax.numpy as jnp
from jax import lax
from jax.experimental import pallas as pl
from jax.experimental.pallas import tpu as pltpu
from jax.experimental.pallas import tpu_sc as plsc

_B, _L, _C = 8, 16384, 128
_S = 256
_N = _B * _L
_NC, _NS = 2, 16
_CHUNK = 8192
_PASSES_PER_SC = _N // _CHUNK // _NC
_BLK = 128
_JUNK = _CHUNK
_PAD = 2 * _BLK


def _sc_body(feats, permr, segr, metar, zrows, out,
             acc, zv,
             pidx0, sidx0, dloc0, rows0,
             pidx1, sidx1, dloc1, rows1,
             mvec, semi0, semg0, sems0, semi1, semg1, sems1):
    c = lax.axis_index("c")
    t = lax.axis_index("s")
    rows_per_tile = _CHUNK // _NS

    pltpu.sync_copy(zrows, zv)
    pltpu.sync_copy(metar.at[c], mvec)
    mv = mvec[...]

    bufs = ((pidx0, sidx0, dloc0, rows0, semi0, semg0, sems0),
            (pidx1, sidx1, dloc1, rows1, semi1, semg1, sems1))

    for p in range(_PASSES_PER_SC):
        base = (c * _PASSES_PER_SC + p) * _CHUNK

        for q in range(rows_per_tile // _BLK):
            pltpu.sync_copy(zv, acc.at[pl.ds(t * rows_per_tile + q * _BLK, _BLK)])

        @pl.when(t == _NS - 1)
        def _zero_junk():
            pltpu.sync_copy(zv.at[pl.ds(0, 8)], acc.at[pl.ds(_CHUNK, 8)])

        plsc.subcore_barrier()

        pos0 = mv[p]
        nblk = mv[_PASSES_PER_SC + p]

        nsteps = jnp.maximum(nblk - t + _NS - 1, 0) // _NS

        def _blockpos(ib):
            return pl.multiple_of(pos0 + (t + ib * _NS) * _BLK, 8)

        def _issue_idx(ib, parity):
            pi, si, _, _, smi, _, _ = bufs[parity]
            pos = _blockpos(ib)
            pltpu.async_copy(permr.at[pl.ds(pos, _BLK)], pi, smi)
            pltpu.async_copy(segr.at[pl.ds(pos, _BLK)], si, smi)

        def _wait_idx(ib, parity):
            pi, si, _, _, smi, _, _ = bufs[parity]
            pos = _blockpos(ib)
            pltpu.make_async_copy(permr.at[pl.ds(pos, _BLK)], pi, smi).wait()
            pltpu.make_async_copy(segr.at[pl.ds(pos, _BLK)], si, smi).wait()

        for parity in (0, 1):
            @pl.when(parity < nsteps)
            def _pro(parity=parity):
                _issue_idx(jnp.int32(parity), parity)

        def _iter(i, carry):
            for parity in (0, 1):
                ib = 2 * i + parity

                @pl.when(ib < nsteps)
                def _do(ib=ib, parity=parity):
                    pi, si, dl, rw, smi, smg, sms = bufs[parity]
                    _wait_idx(ib, parity)
                    @pl.when(ib >= 2)
                    def _ws():
                        pltpu.make_async_copy(rw, acc.at[dl], sms).wait()
                    for k in range(_BLK // 16):
                        sv = si[pl.ds(k * 16, 16)] - base
                        ok = (sv >= 0) & (sv < _CHUNK)
                        dl[pl.ds(k * 16, 16)] = jnp.where(ok, sv, jnp.int32(_JUNK))
                    g = pltpu.async_copy(feats.at[pi], rw, smg)
                    g.wait()

                    @pl.when(ib + 2 < nsteps)
                    def _ni():
                        _issue_idx(ib + 2, parity)
                    pltpu.async_copy(rw, acc.at[dl], sms, add=True)
            return carry

        lax.fori_loop(0, (nsteps + 1) // 2, _iter, jnp.int32(0))

        for parity in (0, 1):
            @pl.when(nsteps > parity)
            def _drain(parity=parity):
                _, _, dl, rw, _, _, sms = bufs[parity]
                pltpu.make_async_copy(rw, acc.at[dl], sms).wait()

        plsc.subcore_barrier()

        obase = pl.multiple_of(base + t * rows_per_tile, 8)
        pltpu.sync_copy(acc.at[pl.ds(t * rows_per_tile, rows_per_tile)],
                        out.at[pl.ds(obase, rows_per_tile)])
        plsc.subcore_barrier()


_mesh = plsc.VectorSubcoreMesh(core_axis_name="c", subcore_axis_name="s")

_sc_call = pl.kernel(
    _sc_body,
    out_type=jax.ShapeDtypeStruct((_N, _C), jnp.float32),
    mesh=_mesh,
    scratch_types=[
        pltpu.VMEM_SHARED((_CHUNK + 8, _C), jnp.float32),
        pltpu.VMEM((_BLK, _C), jnp.float32),
        pltpu.VMEM((_BLK,), jnp.int32),
        pltpu.VMEM((_BLK,), jnp.int32),
        pltpu.VMEM((_BLK,), jnp.int32),
        pltpu.VMEM((_BLK, _C), jnp.float32),
        pltpu.VMEM((_BLK,), jnp.int32),
        pltpu.VMEM((_BLK,), jnp.int32),
        pltpu.VMEM((_BLK,), jnp.int32),
        pltpu.VMEM((_BLK, _C), jnp.float32),
        pltpu.VMEM((16,), jnp.int32),
        pltpu.SemaphoreType.DMA,
        pltpu.SemaphoreType.DMA,
        pltpu.SemaphoreType.DMA,
        pltpu.SemaphoreType.DMA,
        pltpu.SemaphoreType.DMA,
        pltpu.SemaphoreType.DMA,
    ],
)


@jax.jit
def kernel(coords, features):
    ci = coords.astype(jnp.int32)
    lin = (ci[..., 0] * _S + ci[..., 1]) * _S + ci[..., 2]
    feats = features.reshape(_N, _C)

    local = jnp.broadcast_to(jnp.arange(_L, dtype=jnp.int32), (_B, _L))
    sorted_keys, lperm = lax.sort_key_val(lin, local, dimension=1)
    perm = (lperm + jnp.arange(_B, dtype=jnp.int32)[:, None] * _L).reshape(-1)
    flags = jnp.concatenate([
        jnp.ones((_B, 1), jnp.int32),
        (sorted_keys[:, 1:] != sorted_keys[:, :-1]).astype(jnp.int32),
    ], axis=1).reshape(-1)
    seg = jnp.cumsum(flags, dtype=jnp.int32) - 1

    targets = jnp.arange(_N // _CHUNK + 1, dtype=jnp.int32) * _CHUNK
    bounds = jnp.searchsorted(seg, targets, side="left").astype(jnp.int32)
    pos0 = (bounds[:-1] // 8) * 8
    nblk = jnp.maximum(0, (bounds[1:] - pos0 + _BLK - 1) // _BLK)
    meta = jnp.concatenate([
        pos0.reshape(_NC, _PASSES_PER_SC),
        nblk.reshape(_NC, _PASSES_PER_SC),
    ], axis=1)

    perm_pad = jnp.concatenate([perm, jnp.zeros((_PAD,), jnp.int32)])
    seg_pad = jnp.concatenate([seg, jnp.full((_PAD,), 2 ** 30, jnp.int32)])
    zeros_block = jnp.zeros((_BLK, _C), jnp.float32)

    return _sc_call(feats, perm_pad, seg_pad, meta, zeros_block)


# speedup vs baseline: 5.1360x; 1.0089x over previous
import functools

import jax
import jax.numpy as jnp
from jax import lax
from jax.experimental import pallas as pl
from jax.experimental.pallas import tpu as pltpu
from jax.experimental.pallas import tpu_sc as plsc

_B, _L, _C = 8, 16384, 128
_S = 256
_N = _B * _L
_NC, _NS = 2, 16
_CHUNK = 8192
_PASSES_PER_SC = _N // _CHUNK // _NC
_BLK = 128
_JUNK = _CHUNK
_PAD = 2 * _BLK


def _sc_body(feats, permr, segr, metar, zrows, out,
             acc, zv,
             pidx0, sidx0, dloc0, rows0,
             pidx1, sidx1, dloc1, rows1,
             mvec, semi0, semg0, sems0, semi1, semg1, sems1):
    c = lax.axis_index("c")
    t = lax.axis_index("s")
    rows_per_tile = _CHUNK // _NS

    pltpu.sync_copy(zrows, zv)
    pltpu.sync_copy(metar.at[c], mvec)
    mv = mvec[...]

    bufs = ((pidx0, sidx0, dloc0, rows0, semi0, semg0, sems0),
            (pidx1, sidx1, dloc1, rows1, semi1, semg1, sems1))

    for p in range(_PASSES_PER_SC):
        base = (c * _PASSES_PER_SC + p) * _CHUNK

        zcps = [
            pltpu.async_copy(
                zv, acc.at[pl.ds(t * rows_per_tile + q * _BLK, _BLK)], semg0)
            for q in range(rows_per_tile // _BLK)
        ]
        for zcp in zcps:
            zcp.wait()

        plsc.subcore_barrier()

        pos0 = mv[p]
        nblk = mv[_PASSES_PER_SC + p]

        nsteps = jnp.maximum(nblk - t + _NS - 1, 0) // _NS

        def _blockpos(ib):
            return pl.multiple_of(pos0 + (t + ib * _NS) * _BLK, 8)

        def _issue_idx(ib, parity):
            pi, si, _, _, smi, _, _ = bufs[parity]
            pos = _blockpos(ib)
            pltpu.async_copy(permr.at[pl.ds(pos, _BLK)], pi, smi)
            pltpu.async_copy(segr.at[pl.ds(pos, _BLK)], si, smi)

        def _wait_idx(ib, parity):
            pi, si, _, _, smi, _, _ = bufs[parity]
            pos = _blockpos(ib)
            pltpu.make_async_copy(permr.at[pl.ds(pos, _BLK)], pi, smi).wait()
            pltpu.make_async_copy(segr.at[pl.ds(pos, _BLK)], si, smi).wait()

        for parity in (0, 1):
            @pl.when(parity < nsteps)
            def _pro(parity=parity):
                _issue_idx(jnp.int32(parity), parity)

        def _iter(i, carry):
            for parity in (0, 1):
                ib = 2 * i + parity

                @pl.when(ib < nsteps)
                def _do(ib=ib, parity=parity):
                    pi, si, dl, rw, smi, smg, sms = bufs[parity]
                    _wait_idx(ib, parity)
                    @pl.when(ib >= 2)
                    def _ws():
                        pltpu.make_async_copy(rw, acc.at[dl], sms).wait()
                    for k in range(_BLK // 16):
                        sv = si[pl.ds(k * 16, 16)] - base
                        ok = (sv >= 0) & (sv < _CHUNK)
                        dl[pl.ds(k * 16, 16)] = jnp.where(ok, sv, jnp.int32(_JUNK))
                    g = pltpu.async_copy(feats.at[pi], rw, smg)
                    g.wait()

                    @pl.when(ib + 2 < nsteps)
                    def _ni():
                        _issue_idx(ib + 2, parity)
                    pltpu.async_copy(rw, acc.at[dl], sms, add=True)
            return carry

        lax.fori_loop(0, (nsteps + 1) // 2, _iter, jnp.int32(0))

        for parity in (0, 1):
            @pl.when(nsteps > parity)
            def _drain(parity=parity):
                _, _, dl, rw, _, _, sms = bufs[parity]
                pltpu.make_async_copy(rw, acc.at[dl], sms).wait()

        plsc.subcore_barrier()

        obase = pl.multiple_of(base + t * rows_per_tile, 8)
        pltpu.sync_copy(acc.at[pl.ds(t * rows_per_tile, rows_per_tile)],
                        out.at[pl.ds(obase, rows_per_tile)])
        plsc.subcore_barrier()


_mesh = plsc.VectorSubcoreMesh(core_axis_name="c", subcore_axis_name="s")

_sc_call = pl.kernel(
    _sc_body,
    out_type=jax.ShapeDtypeStruct((_N, _C), jnp.float32),
    mesh=_mesh,
    scratch_types=[
        pltpu.VMEM_SHARED((_CHUNK + 8, _C), jnp.float32),
        pltpu.VMEM((_BLK, _C), jnp.float32),
        pltpu.VMEM((_BLK,), jnp.int32),
        pltpu.VMEM((_BLK,), jnp.int32),
        pltpu.VMEM((_BLK,), jnp.int32),
        pltpu.VMEM((_BLK, _C), jnp.float32),
        pltpu.VMEM((_BLK,), jnp.int32),
        pltpu.VMEM((_BLK,), jnp.int32),
        pltpu.VMEM((_BLK,), jnp.int32),
        pltpu.VMEM((_BLK, _C), jnp.float32),
        pltpu.VMEM((16,), jnp.int32),
        pltpu.SemaphoreType.DMA,
        pltpu.SemaphoreType.DMA,
        pltpu.SemaphoreType.DMA,
        pltpu.SemaphoreType.DMA,
        pltpu.SemaphoreType.DMA,
        pltpu.SemaphoreType.DMA,
    ],
)


@jax.jit
def kernel(coords, features):
    ci = coords.astype(jnp.int32)
    lin = (ci[..., 0] * _S + ci[..., 1]) * _S + ci[..., 2]
    feats = features.reshape(_N, _C)

    local = jnp.broadcast_to(jnp.arange(_L, dtype=jnp.int32), (_B, _L))
    sorted_keys, lperm = lax.sort_key_val(lin, local, dimension=1)
    perm = (lperm + jnp.arange(_B, dtype=jnp.int32)[:, None] * _L).reshape(-1)
    flags = jnp.concatenate([
        jnp.ones((_B, 1), jnp.int32),
        (sorted_keys[:, 1:] != sorted_keys[:, :-1]).astype(jnp.int32),
    ], axis=1).reshape(-1)
    seg = jnp.cumsum(flags, dtype=jnp.int32) - 1

    targets = jnp.arange(_N // _CHUNK + 1, dtype=jnp.int32) * _CHUNK
    bounds = jnp.searchsorted(seg, targets, side="left").astype(jnp.int32)
    pos0 = (bounds[:-1] // 8) * 8
    nblk = jnp.maximum(0, (bounds[1:] - pos0 + _BLK - 1) // _BLK)
    meta = jnp.concatenate([
        pos0.reshape(_NC, _PASSES_PER_SC),
        nblk.reshape(_NC, _PASSES_PER_SC),
    ], axis=1)

    perm_pad = jnp.concatenate([perm, jnp.zeros((_PAD,), jnp.int32)])
    seg_pad = jnp.concatenate([seg, jnp.full((_PAD,), 2 ** 30, jnp.int32)])
    zeros_block = jnp.zeros((_BLK, _C), jnp.float32)

    return _sc_call(feats, perm_pad, seg_pad, meta, zeros_block)


# speedup vs baseline: 5.2233x; 1.0170x over previous
import functools

import jax
import jax.numpy as jnp
from jax import lax
from jax.experimental import pallas as pl
from jax.experimental.pallas import tpu as pltpu
from jax.experimental.pallas import tpu_sc as plsc

_B, _L, _C = 8, 16384, 128
_S = 256
_N = _B * _L
_NC, _NS = 2, 16
_CHUNK = 8192
_PASSES_PER_SC = _N // _CHUNK // _NC
_BLK = 128
_JUNK = _CHUNK
_PAD = 2 * _BLK


def _sc_body(feats, permr, segr, metar, zrows, out,
             acc, zv,
             pidx0, sidx0, dloc0, rows0,
             pidx1, sidx1, dloc1, rows1,
             mvec, semi0, semg0, sems0, semi1, semg1, sems1):
    c = lax.axis_index("c")
    t = lax.axis_index("s")
    rows_per_tile = _CHUNK // _NS

    pltpu.sync_copy(zrows, zv)
    pltpu.sync_copy(metar.at[c], mvec)
    mv = mvec[...]

    bufs = ((pidx0, sidx0, dloc0, rows0, semi0, semg0, sems0),
            (pidx1, sidx1, dloc1, rows1, semi1, semg1, sems1))

    for p in range(_PASSES_PER_SC):
        base = (c * _PASSES_PER_SC + p) * _CHUNK

        pos0 = mv[p]
        nblk = mv[_PASSES_PER_SC + p]

        nsteps = jnp.maximum(nblk - t + _NS - 1, 0) // _NS

        def _blockpos(ib):
            return pl.multiple_of(pos0 + (t + ib * _NS) * _BLK, 8)

        def _issue_idx(ib, parity):
            pi, si, _, _, smi, _, _ = bufs[parity]
            pos = _blockpos(ib)
            pltpu.async_copy(permr.at[pl.ds(pos, _BLK)], pi, smi)
            pltpu.async_copy(segr.at[pl.ds(pos, _BLK)], si, smi)

        def _wait_idx(ib, parity):
            pi, si, _, _, smi, _, _ = bufs[parity]
            pos = _blockpos(ib)
            pltpu.make_async_copy(permr.at[pl.ds(pos, _BLK)], pi, smi).wait()
            pltpu.make_async_copy(segr.at[pl.ds(pos, _BLK)], si, smi).wait()

        for parity in (0, 1):
            @pl.when(parity < nsteps)
            def _pro(parity=parity):
                _issue_idx(jnp.int32(parity), parity)

        zcps = [
            pltpu.async_copy(
                zv, acc.at[pl.ds(t * rows_per_tile + q * _BLK, _BLK)], semg0)
            for q in range(rows_per_tile // _BLK)
        ]
        for zcp in zcps:
            zcp.wait()

        plsc.subcore_barrier()

        def _iter(i, carry):
            for parity in (0, 1):
                ib = 2 * i + parity

                @pl.when(ib < nsteps)
                def _do(ib=ib, parity=parity):
                    pi, si, dl, rw, smi, smg, sms = bufs[parity]
                    _wait_idx(ib, parity)
                    @pl.when(ib >= 2)
                    def _ws():
                        pltpu.make_async_copy(rw, acc.at[dl], sms).wait()
                    for k in range(_BLK // 16):
                        sv = si[pl.ds(k * 16, 16)] - base
                        ok = (sv >= 0) & (sv < _CHUNK)
                        dl[pl.ds(k * 16, 16)] = jnp.where(ok, sv, jnp.int32(_JUNK))
                    g = pltpu.async_copy(feats.at[pi], rw, smg)
                    g.wait()

                    @pl.when(ib + 2 < nsteps)
                    def _ni():
                        _issue_idx(ib + 2, parity)
                    pltpu.async_copy(rw, acc.at[dl], sms, add=True)
            return carry

        lax.fori_loop(0, (nsteps + 1) // 2, _iter, jnp.int32(0))

        for parity in (0, 1):
            @pl.when(nsteps > parity)
            def _drain(parity=parity):
                _, _, dl, rw, _, _, sms = bufs[parity]
                pltpu.make_async_copy(rw, acc.at[dl], sms).wait()

        plsc.subcore_barrier()

        obase = pl.multiple_of(base + t * rows_per_tile, 8)
        pltpu.sync_copy(acc.at[pl.ds(t * rows_per_tile, rows_per_tile)],
                        out.at[pl.ds(obase, rows_per_tile)])
        plsc.subcore_barrier()


_mesh = plsc.VectorSubcoreMesh(core_axis_name="c", subcore_axis_name="s")

_sc_call = pl.kernel(
    _sc_body,
    out_type=jax.ShapeDtypeStruct((_N, _C), jnp.float32),
    mesh=_mesh,
    scratch_types=[
        pltpu.VMEM_SHARED((_CHUNK + 8, _C), jnp.float32),
        pltpu.VMEM((_BLK, _C), jnp.float32),
        pltpu.VMEM((_BLK,), jnp.int32),
        pltpu.VMEM((_BLK,), jnp.int32),
        pltpu.VMEM((_BLK,), jnp.int32),
        pltpu.VMEM((_BLK, _C), jnp.float32),
        pltpu.VMEM((_BLK,), jnp.int32),
        pltpu.VMEM((_BLK,), jnp.int32),
        pltpu.VMEM((_BLK,), jnp.int32),
        pltpu.VMEM((_BLK, _C), jnp.float32),
        pltpu.VMEM((16,), jnp.int32),
        pltpu.SemaphoreType.DMA,
        pltpu.SemaphoreType.DMA,
        pltpu.SemaphoreType.DMA,
        pltpu.SemaphoreType.DMA,
        pltpu.SemaphoreType.DMA,
        pltpu.SemaphoreType.DMA,
    ],
)


@jax.jit
def kernel(coords, features):
    ci = coords.astype(jnp.int32)
    lin = (ci[..., 0] * _S + ci[..., 1]) * _S + ci[..., 2]
    feats = features.reshape(_N, _C)

    local = jnp.broadcast_to(jnp.arange(_L, dtype=jnp.int32), (_B, _L))
    sorted_keys, lperm = lax.sort_key_val(lin, local, dimension=1)
    perm = (lperm + jnp.arange(_B, dtype=jnp.int32)[:, None] * _L).reshape(-1)
    flags = jnp.concatenate([
        jnp.ones((_B, 1), jnp.int32),
        (sorted_keys[:, 1:] != sorted_keys[:, :-1]).astype(jnp.int32),
    ], axis=1).reshape(-1)
    seg = jnp.cumsum(flags, dtype=jnp.int32) - 1

    targets = jnp.arange(_N // _CHUNK + 1, dtype=jnp.int32) * _CHUNK
    bounds = jnp.searchsorted(seg, targets, side="left").astype(jnp.int32)
    pos0 = (bounds[:-1] // 8) * 8
    nblk = jnp.maximum(0, (bounds[1:] - pos0 + _BLK - 1) // _BLK)
    meta = jnp.concatenate([
        pos0.reshape(_NC, _PASSES_PER_SC),
        nblk.reshape(_NC, _PASSES_PER_SC),
    ], axis=1)

    perm_pad = jnp.concatenate([perm, jnp.zeros((_PAD,), jnp.int32)])
    seg_pad = jnp.concatenate([seg, jnp.full((_PAD,), 2 ** 30, jnp.int32)])
    zeros_block = jnp.zeros((_BLK, _C), jnp.float32)

    return _sc_call(feats, perm_pad, seg_pad, meta, zeros_block)
